# baseline (speedup 1.0000x reference)
import jax
import jax.numpy as jnp
from jax.experimental import pallas as pl
from jax.experimental.pallas import tpu as pltpu
from kernel_sc import sc_nn

K_SC = 256


def _nn_kernel(s1_ref, s2t_ref, out_ref):
    x = s1_ref[0]
    yt = s2t_ref[0]
    inner = jnp.dot(x, yt, preferred_element_type=jnp.float32)
    x_sq = jnp.sum(x * x, axis=1, keepdims=True)
    y_sq = jnp.sum(yt * yt, axis=0, keepdims=True)
    dist = (x_sq + y_sq) - 2.0 * inner
    idx = jnp.argmin(dist, axis=1).astype(jnp.int32)
    out_ref[0, 0, :] = idx


def _impl(S1, S2):
    B, N, D = S1.shape
    M = S2.shape[1]
    TN = 256
    nb = N // TN
    S2t = jnp.transpose(S2, (0, 2, 1))

    out = pl.pallas_call(
        _nn_kernel,
        grid=(B, nb),
        in_specs=[
            pl.BlockSpec((1, TN, D), lambda b, i: (b, i, 0)),
            pl.BlockSpec((1, D, M), lambda b, i: (b, 0, 0)),
        ],
        out_specs=pl.BlockSpec((1, 1, TN), lambda b, i: (b * nb + i, 0, 0)),
        out_shape=jax.ShapeDtypeStruct((B * nb, 1, TN), jnp.int32),
        compiler_params=pltpu.CompilerParams(
            dimension_semantics=("parallel", "parallel"),
        ),
    )(S1, S2t)
    return out.reshape(B, N)


def kernel(S1, S2):
    out_sc = sc_nn(S1[:, :K_SC], S2)
    out_tc = _impl(S1[:, K_SC:], S2)
    return jnp.concatenate([out_sc, out_tc], axis=1).astype(jnp.int64)


# speedup vs baseline: 1.1697x; 1.1697x over previous
import jax
import jax.numpy as jnp
import numpy as np
from jax.experimental import pallas as pl
from jax.experimental.pallas import tpu as pltpu
from jax.sharding import Mesh, PartitionSpec as P
from jax.experimental.shard_map import shard_map


def _nn_kernel(s1_ref, s2t_ref, out_ref):
    x = s1_ref[0]
    yt = s2t_ref[0]
    inner = jnp.dot(x, yt, preferred_element_type=jnp.float32)
    x_sq = jnp.sum(x * x, axis=1, keepdims=True)
    y_sq = jnp.sum(yt * yt, axis=0, keepdims=True)
    dist = (x_sq + y_sq) - 2.0 * inner
    idx = jnp.argmin(dist, axis=1).astype(jnp.int32)
    out_ref[0, 0, :] = idx


def _impl(S1, S2):
    B, N, D = S1.shape
    M = S2.shape[1]
    TN = 256
    nb = N // TN
    S2t = jnp.transpose(S2, (0, 2, 1))

    out = pl.pallas_call(
        _nn_kernel,
        grid=(B, nb),
        in_specs=[
            pl.BlockSpec((1, TN, D), lambda b, i: (b, i, 0)),
            pl.BlockSpec((1, D, M), lambda b, i: (b, 0, 0)),
        ],
        out_specs=pl.BlockSpec((1, 1, TN), lambda b, i: (b * nb + i, 0, 0)),
        out_shape=jax.ShapeDtypeStruct((B * nb, 1, TN), jnp.int32),
        compiler_params=pltpu.CompilerParams(
            dimension_semantics=("parallel", "parallel"),
        ),
    )(S1, S2t)
    return out.reshape(B, N)


def kernel(S1, S2):
    return _impl(S1, S2).astype(jnp.int64)


# speedup vs baseline: 1.3277x; 1.1351x over previous
import jax
import jax.numpy as jnp
from jax.experimental import pallas as pl
from jax.experimental.pallas import tpu as pltpu


def _nn_kernel(s1_ref, s2t_ref, out_ref):
    x = s1_ref[0]
    yt = s2t_ref[0]
    inner = jnp.dot(x, yt, preferred_element_type=jnp.float32)
    x_sq = jnp.sum(x * x, axis=1, keepdims=True)
    y_sq = jnp.sum(yt * yt, axis=0, keepdims=True)
    dist = (x_sq + y_sq) - 2.0 * inner
    idx = jnp.argmin(dist, axis=1).astype(jnp.int32)
    out_ref[0, 0, :] = idx


def kernel(S1, S2):
    B, N, D = S1.shape
    M = S2.shape[1]
    TN = 512
    nb = N // TN
    S2t = jnp.transpose(S2, (0, 2, 1))

    out = pl.pallas_call(
        _nn_kernel,
        grid=(B, nb),
        in_specs=[
            pl.BlockSpec((1, TN, D), lambda b, i: (b, i, 0)),
            pl.BlockSpec((1, D, M), lambda b, i: (b, 0, 0)),
        ],
        out_specs=pl.BlockSpec((1, 1, TN), lambda b, i: (b * nb + i, 0, 0)),
        out_shape=jax.ShapeDtypeStruct((B * nb, 1, TN), jnp.int32),
        compiler_params=pltpu.CompilerParams(
            dimension_semantics=("parallel", "parallel"),
        ),
    )(S1, S2t)
    return out.reshape(B, N).astype(jnp.int64)


# speedup vs baseline: 1.3827x; 1.0414x over previous
import jax
import jax.numpy as jnp
from jax.experimental import pallas as pl
from jax.experimental.pallas import tpu as pltpu


def _nn_kernel(s1_ref, s2t_ref, out_ref):
    x = s1_ref[0]
    yt = s2t_ref[0]
    inner = jnp.dot(x, yt, preferred_element_type=jnp.float32)
    x_sq = jnp.sum(x * x, axis=1, keepdims=True)
    y_sq = jnp.sum(yt * yt, axis=0, keepdims=True)
    dist = (x_sq + y_sq) - 2.0 * inner
    idx = jnp.argmin(dist, axis=1).astype(jnp.int32)
    out_ref[0, 0, :] = idx


def kernel(S1, S2):
    B, N, D = S1.shape
    M = S2.shape[1]
    TN = 1024
    nb = N // TN
    S2t = jnp.transpose(S2, (0, 2, 1))

    out = pl.pallas_call(
        _nn_kernel,
        grid=(B, nb),
        in_specs=[
            pl.BlockSpec((1, TN, D), lambda b, i: (b, i, 0)),
            pl.BlockSpec((1, D, M), lambda b, i: (b, 0, 0)),
        ],
        out_specs=pl.BlockSpec((1, 1, TN), lambda b, i: (b * nb + i, 0, 0)),
        out_shape=jax.ShapeDtypeStruct((B * nb, 1, TN), jnp.int32),
        compiler_params=pltpu.CompilerParams(
            dimension_semantics=("parallel", "parallel"),
        ),
    )(S1, S2t)
    return out.reshape(B, N).astype(jnp.int64)
